# Initial kernel scaffold; baseline (speedup 1.0000x reference)
import jax
import jax.numpy as jnp
from jax.experimental import pallas as pl


def kernel(x, edge_index, W_in, b_in, W1, att_src1, att_dst1, b1, Wr1, br1, g1, be1, W2, att_src2, att_dst2, b2, Wr2, br2, g2, be2, Wout, bout):
    raise NotImplementedError("write your pallas kernel here")


# speedup vs baseline: 20.9074x; 20.9074x over previous
import functools

import jax
import jax.numpy as jnp
from jax import lax
from jax.experimental import pallas as pl
from jax.experimental.pallas import tpu as pltpu
from jax.experimental.pallas import tpu_sc as plsc

N = 10000
NPAD = 10240
E = 160000
ETOT = E + N
EPAD = 172032
BLK = 512
NB = NPAD // BLK
CB = 128
PER_TEC1 = EPAD // 16
PER_TEC2 = EPAD // 32
NCH1 = PER_TEC1 // CB
NCH2 = PER_TEC2 // CB
ROWS_PER_TEC = NPAD // 16

_i32 = jnp.int32
_f32 = jnp.float32


def _lrelu(x):
    return jnp.maximum(x, 0.2 * x)


def _stage_a_body(x_ref, consts_ref, attc_ref, h1_ref, res1_ref, att1_ref, bmax_ref):
    xv = x_ref[:, :]
    h1_ref[:, :] = xv * consts_ref[0:1, :] + consts_ref[1:2, :]
    res1_ref[:, :] = xv * consts_ref[2:3, :] + consts_ref[3:4, :]
    att = xv * attc_ref[0:1, :] + attc_ref[1:2, :]
    att1_ref[:, :] = att
    bmax_ref[0, :, :] = jnp.max(att, axis=0, keepdims=True)


def _stage_a(xp, consts, attc):
    return pl.pallas_call(
        _stage_a_body,
        grid=(NB,),
        in_specs=[
            pl.BlockSpec((BLK, 1), lambda i: (i, 0)),
            pl.BlockSpec((4, 256), lambda i: (0, 0)),
            pl.BlockSpec((2, 16), lambda i: (0, 0)),
        ],
        out_specs=[
            pl.BlockSpec((BLK, 256), lambda i: (i, 0)),
            pl.BlockSpec((BLK, 256), lambda i: (i, 0)),
            pl.BlockSpec((BLK, 16), lambda i: (i, 0)),
            pl.BlockSpec((1, 1, 16), lambda i: (i, 0, 0)),
        ],
        out_shape=[
            jax.ShapeDtypeStruct((NPAD, 256), _f32),
            jax.ShapeDtypeStruct((NPAD, 256), _f32),
            jax.ShapeDtypeStruct((NPAD, 16), _f32),
            jax.ShapeDtypeStruct((NB, 1, 16), _f32),
        ],
    )(xp, consts, attc)


def _stage_c_body(acca_ref, accb_ref, denr_ref, res1_ref, cvec_ref, w2_ref,
                  wr2_ref, c32_ref, g2_ref, res2_ref, att2_ref, bmax2_ref):
    acc = jnp.concatenate([acca_ref[:, :], accb_ref[:, :]], axis=1)
    o = acc / (denr_ref[:, :] + 1e-16) + cvec_ref[0:1, :]
    o = jnp.where(o > 0, o, jnp.exp(jnp.minimum(o, 0.0)) - 1.0)
    t = o + res1_ref[:, :]
    m = jnp.mean(t, axis=-1, keepdims=True)
    v = jnp.mean((t - m) ** 2, axis=-1, keepdims=True)
    h = (t - m) * lax.rsqrt(v + 1e-5) * cvec_ref[1:2, :] + cvec_ref[2:3, :]
    g2 = jnp.dot(h, w2_ref[:, :], preferred_element_type=_f32)
    g2_ref[:, :] = g2
    res2_ref[:, :] = jnp.dot(h, wr2_ref[:, :], preferred_element_type=_f32) + c32_ref[2:3, :]
    asrc2 = jnp.sum(g2 * c32_ref[0:1, :], axis=-1, keepdims=True)
    adst2 = jnp.sum(g2 * c32_ref[1:2, :], axis=-1, keepdims=True)
    lane = lax.broadcasted_iota(_i32, (BLK, 8), 1)
    att2 = jnp.where(lane == 0, asrc2, jnp.where(lane == 1, adst2, 0.0))
    att2_ref[:, :] = att2
    bmax2_ref[0, :, :] = jnp.max(att2, axis=0, keepdims=True)


def _stage_c(acca, accb, denr, res1, cvec, w2, wr2, c32):
    return pl.pallas_call(
        _stage_c_body,
        grid=(NB,),
        in_specs=[
            pl.BlockSpec((BLK, 128), lambda i: (i, 0)),
            pl.BlockSpec((BLK, 128), lambda i: (i, 0)),
            pl.BlockSpec((BLK, 256), lambda i: (i, 0)),
            pl.BlockSpec((BLK, 256), lambda i: (i, 0)),
            pl.BlockSpec((3, 256), lambda i: (0, 0)),
            pl.BlockSpec((256, 32), lambda i: (0, 0)),
            pl.BlockSpec((256, 32), lambda i: (0, 0)),
            pl.BlockSpec((3, 32), lambda i: (0, 0)),
        ],
        out_specs=[
            pl.BlockSpec((BLK, 32), lambda i: (i, 0)),
            pl.BlockSpec((BLK, 32), lambda i: (i, 0)),
            pl.BlockSpec((BLK, 8), lambda i: (i, 0)),
            pl.BlockSpec((1, 1, 8), lambda i: (i, 0, 0)),
        ],
        out_shape=[
            jax.ShapeDtypeStruct((NPAD, 32), _f32),
            jax.ShapeDtypeStruct((NPAD, 32), _f32),
            jax.ShapeDtypeStruct((NPAD, 8), _f32),
            jax.ShapeDtypeStruct((NB, 1, 8), _f32),
        ],
    )(acca, accb, denr, res1, cvec, w2, wr2, c32)


def _stage_e_body(a0_ref, a1_ref, d0_ref, d1_ref, res2_ref, c32_ref, psum_ref):
    i = pl.program_id(0)
    den = d0_ref[:, 0:1] + d1_ref[:, 0:1]
    o = (a0_ref[:, :] + a1_ref[:, :]) / (den + 1e-16) + c32_ref[0:1, :] + res2_ref[:, :]
    m = jnp.mean(o, axis=-1, keepdims=True)
    v = jnp.mean((o - m) ** 2, axis=-1, keepdims=True)
    hf = (o - m) * lax.rsqrt(v + 1e-5) * c32_ref[1:2, :] + c32_ref[2:3, :]
    rid = i * BLK + lax.broadcasted_iota(_i32, (BLK, 1), 0)
    hf = jnp.where(rid < N, hf, 0.0)
    psum_ref[0, :, :] = jnp.sum(hf, axis=0, keepdims=True)


def _stage_e(a0, a1, d0, d1, res2, c32):
    return pl.pallas_call(
        _stage_e_body,
        grid=(NB,),
        in_specs=[
            pl.BlockSpec((BLK, 32), lambda i: (i, 0)),
            pl.BlockSpec((BLK, 32), lambda i: (i, 0)),
            pl.BlockSpec((BLK, 16), lambda i: (i, 0)),
            pl.BlockSpec((BLK, 16), lambda i: (i, 0)),
            pl.BlockSpec((BLK, 32), lambda i: (i, 0)),
            pl.BlockSpec((3, 32), lambda i: (0, 0)),
        ],
        out_specs=[pl.BlockSpec((1, 1, 32), lambda i: (i, 0, 0))],
        out_shape=[jax.ShapeDtypeStruct((NB, 1, 32), _f32)],
    )(a0, a1, d0, d1, res2, c32)


def _sc1_scs(srcp, dstp, hcat, att16, crep, zer128, zer16,
             acc_out, den_out, acc_sp, den_sp):
    pass


def _sc1_tec(srcp, dstp, hcat, att16, crep, zer128, zer16,
             acc_out, den_out, acc_sp, den_sp):
    c = lax.axis_index("c")
    s = lax.axis_index("s")
    iota = lax.iota(_i32, 16)
    r0 = s * ROWS_PER_TEC
    ebase = s * PER_TEC1
    coff = c * NPAD

    def inner(srcv, dstv, srcoff, dstoff, abufs, abufd, rowbuf, wrows, crep_vm,
              sem, sema, semb):
        pltpu.sync_copy(crep, crep_vm)
        mv = [plsc.load_gather(crep_vm, [iota + (c * 4 + h) * 16]) for h in range(4)]

        pltpu.sync_copy(zer128.at[pl.ds(r0, ROWS_PER_TEC)], acc_sp.at[pl.ds(r0, ROWS_PER_TEC)])
        pltpu.sync_copy(zer16.at[pl.ds(r0, ROWS_PER_TEC)], den_sp.at[pl.ds(r0, ROWS_PER_TEC)])
        pltpu.sync_copy(zer16.at[pl.ds(0, CB)], wrows)
        plsc.subcore_barrier()

        def chunk(i, _):
            base = ebase + i * CB
            pltpu.sync_copy(srcp.at[pl.ds(base, CB)], srcv)
            pltpu.sync_copy(dstp.at[pl.ds(base, CB)], dstv)
            for g in range(CB // 16):
                srcoff[pl.ds(g * 16, 16)] = srcv[pl.ds(g * 16, 16)] + coff
                dstoff[pl.ds(g * 16, 16)] = dstv[pl.ds(g * 16, 16)] + coff
            gather = pltpu.async_copy(hcat.at[srcoff], rowbuf, sem)
            ga = pltpu.async_copy(att16.at[srcoff], abufs, sema)
            gb = pltpu.async_copy(att16.at[dstoff], abufd, semb)
            ga.wait()
            gb.wait()
            for g in range(CB // 16):
                ridx = iota + g * 16
                for h in range(4):
                    hh = jnp.full((16,), h, _i32)
                    a = plsc.load_gather(abufs, [ridx, hh])
                    b = plsc.load_gather(abufd, [ridx, hh + 4])
                    al = a + b
                    w = jnp.exp(jnp.maximum(al, 0.2 * al) - mv[h])
                    plsc.store_scatter(wrows, [ridx, hh], w)
            gather.wait()

            def scale(e, _):
                ev = jnp.full((16,), 0, _i32) + e
                for h in range(4):
                    wv = plsc.load_gather(wrows, [ev, jnp.full((16,), h, _i32)])
                    for j in range(2):
                        col = iota + (h * 32 + j * 16)
                        rv = plsc.load_gather(rowbuf, [ev, col])
                        plsc.store_scatter(rowbuf, [ev, col], rv * wv)
                return 0

            lax.fori_loop(0, CB, scale, 0)
            pltpu.sync_copy(rowbuf, acc_sp.at[dstv], add=True)
            pltpu.sync_copy(wrows, den_sp.at[dstv], add=True)
            return 0

        lax.fori_loop(0, NCH1, chunk, 0)
        plsc.subcore_barrier()
        pltpu.sync_copy(acc_sp.at[pl.ds(r0, ROWS_PER_TEC)],
                        acc_out.at[pl.ds(coff + r0, ROWS_PER_TEC)])
        pltpu.sync_copy(den_sp.at[pl.ds(r0, ROWS_PER_TEC)],
                        den_out.at[pl.ds(coff + r0, ROWS_PER_TEC)])

    pl.run_scoped(
        inner,
        pltpu.VMEM((CB,), _i32),
        pltpu.VMEM((CB,), _i32),
        pltpu.VMEM((CB,), _i32),
        pltpu.VMEM((CB,), _i32),
        pltpu.VMEM((CB, 16), _f32),
        pltpu.VMEM((CB, 16), _f32),
        pltpu.VMEM((CB, 128), _f32),
        pltpu.VMEM((CB, 16), _f32),
        pltpu.VMEM((128,), _f32),
        pltpu.SemaphoreType.DMA,
        pltpu.SemaphoreType.DMA,
        pltpu.SemaphoreType.DMA,
    )


def _sc_gat1(srcp, dstp, hcat, att16, crep, zer128, zer16):
    vmesh = plsc.VectorSubcoreMesh(core_axis_name="c", subcore_axis_name="s")
    smesh = plsc.ScalarSubcoreMesh(axis_name="c")
    f = pl.kernel(
        [_sc1_scs, _sc1_tec],
        out_type=[
            jax.ShapeDtypeStruct((2 * NPAD, 128), _f32),
            jax.ShapeDtypeStruct((2 * NPAD, 16), _f32),
        ],
        mesh=[smesh, vmesh],
        compiler_params=pltpu.CompilerParams(needs_layout_passes=False, use_tc_tiling_on_sc=False),
        scratch_types=[
            pltpu.VMEM_SHARED((NPAD, 128), _f32),
            pltpu.VMEM_SHARED((NPAD, 16), _f32),
        ],
    )
    return f(srcp, dstp, hcat, att16, crep, zer128, zer16)


def _sc2_scs(srcp, dstp, g2tab, att2, c2rep, zer32, zer16,
             acc_out, den_out, acc_sp, den_sp):
    pass


def _sc2_tec(srcp, dstp, g2tab, att2, c2rep, zer32, zer16,
             acc_out, den_out, acc_sp, den_sp):
    c = lax.axis_index("c")
    s = lax.axis_index("s")
    iota = lax.iota(_i32, 16)
    zero16 = jnp.full((16,), 0, _i32)
    r0 = s * ROWS_PER_TEC
    ebase = (c * 16 + s) * PER_TEC2
    coff = c * NPAD

    def inner(att_vm, srcv, dstv, rowbuf, wrows, c2vm, sem):
        pltpu.sync_copy(att2, att_vm)
        pltpu.sync_copy(c2rep, c2vm)
        mv = c2vm[pl.ds(0, 16)]

        pltpu.sync_copy(zer32.at[pl.ds(r0, ROWS_PER_TEC)], acc_sp.at[pl.ds(r0, ROWS_PER_TEC)])
        pltpu.sync_copy(zer16.at[pl.ds(r0, ROWS_PER_TEC)], den_sp.at[pl.ds(r0, ROWS_PER_TEC)])
        pltpu.sync_copy(zer16.at[pl.ds(0, CB)], wrows)
        plsc.subcore_barrier()

        def chunk(i, _):
            base = ebase + i * CB
            pltpu.sync_copy(srcp.at[pl.ds(base, CB)], srcv)
            pltpu.sync_copy(dstp.at[pl.ds(base, CB)], dstv)
            gather = pltpu.async_copy(g2tab.at[srcv], rowbuf, sem)
            for g in range(CB // 16):
                sidx = srcv[pl.ds(g * 16, 16)]
                didx = dstv[pl.ds(g * 16, 16)]
                a = plsc.load_gather(att_vm, [sidx, jnp.full((16,), 0, _i32)])
                b = plsc.load_gather(att_vm, [didx, jnp.full((16,), 1, _i32)])
                al = a + b
                w = jnp.exp(jnp.maximum(al, 0.2 * al) - mv)
                plsc.store_scatter(wrows, [iota + g * 16, jnp.full((16,), 0, _i32)], w)
            gather.wait()

            def scale(e, _):
                ev = zero16 + e
                wv = plsc.load_gather(wrows, [ev, zero16])
                for j in range(2):
                    col = iota + j * 16
                    rv = plsc.load_gather(rowbuf, [ev, col])
                    plsc.store_scatter(rowbuf, [ev, col], rv * wv)
                return 0

            lax.fori_loop(0, CB, scale, 0)
            pltpu.sync_copy(rowbuf, acc_sp.at[dstv], add=True)
            pltpu.sync_copy(wrows, den_sp.at[dstv], add=True)
            return 0

        lax.fori_loop(0, NCH2, chunk, 0)
        plsc.subcore_barrier()
        pltpu.sync_copy(acc_sp.at[pl.ds(r0, ROWS_PER_TEC)],
                        acc_out.at[pl.ds(coff + r0, ROWS_PER_TEC)])
        pltpu.sync_copy(den_sp.at[pl.ds(r0, ROWS_PER_TEC)],
                        den_out.at[pl.ds(coff + r0, ROWS_PER_TEC)])

    pl.run_scoped(
        inner,
        pltpu.VMEM((NPAD, 8), _f32),
        pltpu.VMEM((CB,), _i32),
        pltpu.VMEM((CB,), _i32),
        pltpu.VMEM((CB, 32), _f32),
        pltpu.VMEM((CB, 16), _f32),
        pltpu.VMEM((16,), _f32),
        pltpu.SemaphoreType.DMA,
    )


def _sc_gat2(srcp, dstp, g2tab, att2, c2rep, zer32, zer16):
    vmesh = plsc.VectorSubcoreMesh(core_axis_name="c", subcore_axis_name="s")
    smesh = plsc.ScalarSubcoreMesh(axis_name="c")
    f = pl.kernel(
        [_sc2_scs, _sc2_tec],
        out_type=[
            jax.ShapeDtypeStruct((2 * NPAD, 32), _f32),
            jax.ShapeDtypeStruct((2 * NPAD, 16), _f32),
        ],
        mesh=[smesh, vmesh],
        compiler_params=pltpu.CompilerParams(needs_layout_passes=False, use_tc_tiling_on_sc=False),
        scratch_types=[
            pltpu.VMEM_SHARED((NPAD, 32), _f32),
            pltpu.VMEM_SHARED((NPAD, 16), _f32),
        ],
    )
    return f(srcp, dstp, g2tab, att2, c2rep, zer32, zer16)


def kernel(x, edge_index, W_in, b_in, W1, att_src1, att_dst1, b1, Wr1, br1, g1, be1,
           W2, att_src2, att_dst2, b2, Wr2, br2, g2, be2, Wout, bout):
    v1 = (W_in @ W1)[0]
    c1 = b_in @ W1
    vr1 = (W_in @ Wr1)[0]
    cr1 = b_in @ Wr1 + br1
    consts = jnp.stack([v1, c1, vr1, cr1])
    ps = (v1.reshape(8, 32) * att_src1[0]).sum(-1)
    qs = (c1.reshape(8, 32) * att_src1[0]).sum(-1)
    pd = (v1.reshape(8, 32) * att_dst1[0]).sum(-1)
    qd = (c1.reshape(8, 32) * att_dst1[0]).sum(-1)
    attc = jnp.stack([jnp.concatenate([ps, pd]), jnp.concatenate([qs, qd])])

    xp = jnp.concatenate([x, jnp.zeros((NPAD - N, 1), _f32)], axis=0)

    loop = jnp.arange(N, dtype=_i32)
    padi = jnp.full((EPAD - ETOT,), N, _i32)
    srcp = jnp.concatenate([edge_index[0].astype(_i32), loop, padi])
    dstp = jnp.concatenate([edge_index[1].astype(_i32), loop, padi])

    h1, res1, att1, bmax = _stage_a(xp, consts, attc)

    ms = jnp.max(bmax, axis=(0, 1))
    cshift = _lrelu(ms[:8] + ms[8:])
    crep = jnp.repeat(cshift, 16)
    hcat = jnp.concatenate([h1[:, :128], h1[:, 128:]], axis=0)
    zpad8 = jnp.zeros((NPAD, 8), _f32)
    att16 = jnp.concatenate([
        jnp.concatenate([att1[:, 0:4], att1[:, 8:12], zpad8], axis=1),
        jnp.concatenate([att1[:, 4:8], att1[:, 12:16], zpad8], axis=1),
    ], axis=0)
    zer128 = jnp.zeros((NPAD, 128), _f32)
    zer16 = jnp.zeros((NPAD, 16), _f32)

    acc1, den1 = _sc_gat1(srcp, dstp, hcat, att16, crep, zer128, zer16)

    den8 = jnp.concatenate([den1[:NPAD, 0:4], den1[NPAD:, 0:4]], axis=1)
    denr = jnp.repeat(den8, 32, axis=1)
    cvec = jnp.stack([b1, g1, be1])
    c32 = jnp.stack([att_src2[0, 0], att_dst2[0, 0], br2])
    g2tab, res2, att2, bmax2 = _stage_c(acc1[:NPAD], acc1[NPAD:], denr, res1,
                                        cvec, W2, Wr2, c32)

    m2 = jnp.max(bmax2, axis=(0, 1))
    c2 = _lrelu(m2[0] + m2[1])
    c2rep = jnp.full((16,), c2, _f32)
    zer32 = jnp.zeros((NPAD, 32), _f32)

    acc2, den2 = _sc_gat2(srcp, dstp, g2tab, att2, c2rep, zer32, zer16)

    c32e = jnp.stack([b2, g2, be2])
    psum = _stage_e(acc2[:NPAD], acc2[NPAD:], den2[:NPAD], den2[NPAD:], res2, c32e)[0]
    pooled = jnp.sum(psum, axis=(0, 1)).reshape(1, 32) / N
    return pooled @ Wout + bout


# speedup vs baseline: 21.1042x; 1.0094x over previous
import functools

import jax
import jax.numpy as jnp
from jax import lax
from jax.experimental import pallas as pl
from jax.experimental.pallas import tpu as pltpu
from jax.experimental.pallas import tpu_sc as plsc

N = 10000
NPAD = 10240
E = 160000
ETOT = E + N
EPAD = 172032
BLK = 512
NB = NPAD // BLK
CB = 128
PER_TEC1 = EPAD // 16
PER_TEC2 = EPAD // 32
NCH1 = PER_TEC1 // CB
NCH2 = PER_TEC2 // CB
ROWS_PER_TEC = NPAD // 16

_i32 = jnp.int32
_f32 = jnp.float32


def _lrelu(x):
    return jnp.maximum(x, 0.2 * x)


def _stage_a_body(x_ref, consts_ref, attc_ref, h1_ref, res1_ref, att1_ref, bmax_ref):
    xv = x_ref[:, :]
    h1_ref[:, :] = xv * consts_ref[0:1, :] + consts_ref[1:2, :]
    res1_ref[:, :] = xv * consts_ref[2:3, :] + consts_ref[3:4, :]
    att = xv * attc_ref[0:1, :] + attc_ref[1:2, :]
    att1_ref[:, :] = att
    bmax_ref[0, :, :] = jnp.max(att, axis=0, keepdims=True)


def _stage_a(xp, consts, attc):
    return pl.pallas_call(
        _stage_a_body,
        grid=(NB,),
        in_specs=[
            pl.BlockSpec((BLK, 1), lambda i: (i, 0)),
            pl.BlockSpec((4, 256), lambda i: (0, 0)),
            pl.BlockSpec((2, 16), lambda i: (0, 0)),
        ],
        out_specs=[
            pl.BlockSpec((BLK, 256), lambda i: (i, 0)),
            pl.BlockSpec((BLK, 256), lambda i: (i, 0)),
            pl.BlockSpec((BLK, 16), lambda i: (i, 0)),
            pl.BlockSpec((1, 1, 16), lambda i: (i, 0, 0)),
        ],
        out_shape=[
            jax.ShapeDtypeStruct((NPAD, 256), _f32),
            jax.ShapeDtypeStruct((NPAD, 256), _f32),
            jax.ShapeDtypeStruct((NPAD, 16), _f32),
            jax.ShapeDtypeStruct((NB, 1, 16), _f32),
        ],
    )(xp, consts, attc)


def _stage_c_body(acca_ref, accb_ref, denr_ref, res1_ref, cvec_ref, w2_ref,
                  wr2_ref, c32_ref, g2_ref, res2_ref, att2_ref, bmax2_ref):
    acc = jnp.concatenate([acca_ref[:, :], accb_ref[:, :]], axis=1)
    o = acc / (denr_ref[:, :] + 1e-16) + cvec_ref[0:1, :]
    o = jnp.where(o > 0, o, jnp.exp(jnp.minimum(o, 0.0)) - 1.0)
    t = o + res1_ref[:, :]
    m = jnp.mean(t, axis=-1, keepdims=True)
    v = jnp.mean((t - m) ** 2, axis=-1, keepdims=True)
    h = (t - m) * lax.rsqrt(v + 1e-5) * cvec_ref[1:2, :] + cvec_ref[2:3, :]
    g2 = jnp.dot(h, w2_ref[:, :], preferred_element_type=_f32)
    g2_ref[:, :] = g2
    res2_ref[:, :] = jnp.dot(h, wr2_ref[:, :], preferred_element_type=_f32) + c32_ref[2:3, :]
    asrc2 = jnp.sum(g2 * c32_ref[0:1, :], axis=-1, keepdims=True)
    adst2 = jnp.sum(g2 * c32_ref[1:2, :], axis=-1, keepdims=True)
    lane = lax.broadcasted_iota(_i32, (BLK, 8), 1)
    att2 = jnp.where(lane == 0, asrc2, jnp.where(lane == 1, adst2, 0.0))
    att2_ref[:, :] = att2
    bmax2_ref[0, :, :] = jnp.max(att2, axis=0, keepdims=True)


def _stage_c(acca, accb, denr, res1, cvec, w2, wr2, c32):
    return pl.pallas_call(
        _stage_c_body,
        grid=(NB,),
        in_specs=[
            pl.BlockSpec((BLK, 128), lambda i: (i, 0)),
            pl.BlockSpec((BLK, 128), lambda i: (i, 0)),
            pl.BlockSpec((BLK, 256), lambda i: (i, 0)),
            pl.BlockSpec((BLK, 256), lambda i: (i, 0)),
            pl.BlockSpec((3, 256), lambda i: (0, 0)),
            pl.BlockSpec((256, 32), lambda i: (0, 0)),
            pl.BlockSpec((256, 32), lambda i: (0, 0)),
            pl.BlockSpec((3, 32), lambda i: (0, 0)),
        ],
        out_specs=[
            pl.BlockSpec((BLK, 32), lambda i: (i, 0)),
            pl.BlockSpec((BLK, 32), lambda i: (i, 0)),
            pl.BlockSpec((BLK, 8), lambda i: (i, 0)),
            pl.BlockSpec((1, 1, 8), lambda i: (i, 0, 0)),
        ],
        out_shape=[
            jax.ShapeDtypeStruct((NPAD, 32), _f32),
            jax.ShapeDtypeStruct((NPAD, 32), _f32),
            jax.ShapeDtypeStruct((NPAD, 8), _f32),
            jax.ShapeDtypeStruct((NB, 1, 8), _f32),
        ],
    )(acca, accb, denr, res1, cvec, w2, wr2, c32)


def _stage_e_body(a0_ref, a1_ref, d0_ref, d1_ref, res2_ref, c32_ref, psum_ref):
    i = pl.program_id(0)
    den = d0_ref[:, 0:1] + d1_ref[:, 0:1]
    o = (a0_ref[:, :] + a1_ref[:, :]) / (den + 1e-16) + c32_ref[0:1, :] + res2_ref[:, :]
    m = jnp.mean(o, axis=-1, keepdims=True)
    v = jnp.mean((o - m) ** 2, axis=-1, keepdims=True)
    hf = (o - m) * lax.rsqrt(v + 1e-5) * c32_ref[1:2, :] + c32_ref[2:3, :]
    rid = i * BLK + lax.broadcasted_iota(_i32, (BLK, 1), 0)
    hf = jnp.where(rid < N, hf, 0.0)
    psum_ref[0, :, :] = jnp.sum(hf, axis=0, keepdims=True)


def _stage_e(a0, a1, d0, d1, res2, c32):
    return pl.pallas_call(
        _stage_e_body,
        grid=(NB,),
        in_specs=[
            pl.BlockSpec((BLK, 32), lambda i: (i, 0)),
            pl.BlockSpec((BLK, 32), lambda i: (i, 0)),
            pl.BlockSpec((BLK, 16), lambda i: (i, 0)),
            pl.BlockSpec((BLK, 16), lambda i: (i, 0)),
            pl.BlockSpec((BLK, 32), lambda i: (i, 0)),
            pl.BlockSpec((3, 32), lambda i: (0, 0)),
        ],
        out_specs=[pl.BlockSpec((1, 1, 32), lambda i: (i, 0, 0))],
        out_shape=[jax.ShapeDtypeStruct((NB, 1, 32), _f32)],
    )(a0, a1, d0, d1, res2, c32)


def _sc1_scs(srcp, dstp, hcat, att16, crep, zer128, zer16,
             acc_out, den_out, acc_sp, den_sp):
    pass


def _sc1_tec(srcp, dstp, hcat, att16, crep, zer128, zer16,
             acc_out, den_out, acc_sp, den_sp):
    c = lax.axis_index("c")
    s = lax.axis_index("s")
    iota = lax.iota(_i32, 16)
    r0 = s * ROWS_PER_TEC
    ebase = s * PER_TEC1
    coff = c * NPAD

    def inner(srcv, dstv, srcoff, dstoff, abufs, abufd, rowbuf, wrows, crep_vm,
              sem, sema, semb):
        pltpu.sync_copy(crep, crep_vm)
        mv = [plsc.load_gather(crep_vm, [iota + (c * 4 + h) * 16]) for h in range(4)]

        pltpu.sync_copy(zer128.at[pl.ds(r0, ROWS_PER_TEC)], acc_sp.at[pl.ds(r0, ROWS_PER_TEC)])
        pltpu.sync_copy(zer16.at[pl.ds(r0, ROWS_PER_TEC)], den_sp.at[pl.ds(r0, ROWS_PER_TEC)])
        pltpu.sync_copy(zer16.at[pl.ds(0, CB)], wrows)
        plsc.subcore_barrier()

        def chunk(i, _):
            base = ebase + i * CB
            pltpu.sync_copy(srcp.at[pl.ds(base, CB)], srcv)
            pltpu.sync_copy(dstp.at[pl.ds(base, CB)], dstv)
            for g in range(CB // 16):
                srcoff[pl.ds(g * 16, 16)] = srcv[pl.ds(g * 16, 16)] + coff
                dstoff[pl.ds(g * 16, 16)] = dstv[pl.ds(g * 16, 16)] + coff
            gather = pltpu.async_copy(hcat.at[srcoff], rowbuf, sem)
            ga = pltpu.async_copy(att16.at[srcoff], abufs, sema)
            gb = pltpu.async_copy(att16.at[dstoff], abufd, semb)
            ga.wait()
            gb.wait()
            for g in range(CB // 16):
                ridx = iota + g * 16
                for h in range(4):
                    hh = jnp.full((16,), h, _i32)
                    a = plsc.load_gather(abufs, [ridx, hh])
                    b = plsc.load_gather(abufd, [ridx, hh + 4])
                    al = a + b
                    w = jnp.exp(jnp.maximum(al, 0.2 * al) - mv[h])
                    plsc.store_scatter(wrows, [ridx, hh], w)
            gather.wait()

            def scale(q, _):
                for u in range(4):
                    ev = jnp.full((16,), 0, _i32) + (q * 4 + u)
                    wv = [plsc.load_gather(wrows, [ev, jnp.full((16,), h, _i32)])
                          for h in range(4)]
                    for h in range(4):
                        for j in range(2):
                            col = iota + (h * 32 + j * 16)
                            rv = plsc.load_gather(rowbuf, [ev, col])
                            plsc.store_scatter(rowbuf, [ev, col], rv * wv[h])
                return 0

            lax.fori_loop(0, CB // 4, scale, 0)
            sc1 = pltpu.async_copy(rowbuf, acc_sp.at[dstv], sema, add=True)
            sc2 = pltpu.async_copy(wrows, den_sp.at[dstv], semb, add=True)
            sc1.wait()
            sc2.wait()
            return 0

        lax.fori_loop(0, NCH1, chunk, 0)
        plsc.subcore_barrier()
        pltpu.sync_copy(acc_sp.at[pl.ds(r0, ROWS_PER_TEC)],
                        acc_out.at[pl.ds(coff + r0, ROWS_PER_TEC)])
        pltpu.sync_copy(den_sp.at[pl.ds(r0, ROWS_PER_TEC)],
                        den_out.at[pl.ds(coff + r0, ROWS_PER_TEC)])

    pl.run_scoped(
        inner,
        pltpu.VMEM((CB,), _i32),
        pltpu.VMEM((CB,), _i32),
        pltpu.VMEM((CB,), _i32),
        pltpu.VMEM((CB,), _i32),
        pltpu.VMEM((CB, 16), _f32),
        pltpu.VMEM((CB, 16), _f32),
        pltpu.VMEM((CB, 128), _f32),
        pltpu.VMEM((CB, 16), _f32),
        pltpu.VMEM((128,), _f32),
        pltpu.SemaphoreType.DMA,
        pltpu.SemaphoreType.DMA,
        pltpu.SemaphoreType.DMA,
    )


def _sc_gat1(srcp, dstp, hcat, att16, crep, zer128, zer16):
    vmesh = plsc.VectorSubcoreMesh(core_axis_name="c", subcore_axis_name="s")
    smesh = plsc.ScalarSubcoreMesh(axis_name="c")
    f = pl.kernel(
        [_sc1_scs, _sc1_tec],
        out_type=[
            jax.ShapeDtypeStruct((2 * NPAD, 128), _f32),
            jax.ShapeDtypeStruct((2 * NPAD, 16), _f32),
        ],
        mesh=[smesh, vmesh],
        compiler_params=pltpu.CompilerParams(needs_layout_passes=False, use_tc_tiling_on_sc=False),
        scratch_types=[
            pltpu.VMEM_SHARED((NPAD, 128), _f32),
            pltpu.VMEM_SHARED((NPAD, 16), _f32),
        ],
    )
    return f(srcp, dstp, hcat, att16, crep, zer128, zer16)


def _sc2_scs(srcp, dstp, g2tab, att2, c2rep, zer32, zer16,
             acc_out, den_out, acc_sp, den_sp):
    pass


def _sc2_tec(srcp, dstp, g2tab, att2, c2rep, zer32, zer16,
             acc_out, den_out, acc_sp, den_sp):
    c = lax.axis_index("c")
    s = lax.axis_index("s")
    iota = lax.iota(_i32, 16)
    zero16 = jnp.full((16,), 0, _i32)
    r0 = s * ROWS_PER_TEC
    ebase = (c * 16 + s) * PER_TEC2
    coff = c * NPAD

    def inner(att_vm, srcv, dstv, rowbuf, wrows, c2vm, sem, sema, semb):
        pltpu.sync_copy(att2, att_vm)
        pltpu.sync_copy(c2rep, c2vm)
        mv = c2vm[pl.ds(0, 16)]

        pltpu.sync_copy(zer32.at[pl.ds(r0, ROWS_PER_TEC)], acc_sp.at[pl.ds(r0, ROWS_PER_TEC)])
        pltpu.sync_copy(zer16.at[pl.ds(r0, ROWS_PER_TEC)], den_sp.at[pl.ds(r0, ROWS_PER_TEC)])
        pltpu.sync_copy(zer16.at[pl.ds(0, CB)], wrows)
        plsc.subcore_barrier()

        def chunk(i, _):
            base = ebase + i * CB
            pltpu.sync_copy(srcp.at[pl.ds(base, CB)], srcv)
            pltpu.sync_copy(dstp.at[pl.ds(base, CB)], dstv)
            gather = pltpu.async_copy(g2tab.at[srcv], rowbuf, sem)
            for g in range(CB // 16):
                sidx = srcv[pl.ds(g * 16, 16)]
                didx = dstv[pl.ds(g * 16, 16)]
                a = plsc.load_gather(att_vm, [sidx, jnp.full((16,), 0, _i32)])
                b = plsc.load_gather(att_vm, [didx, jnp.full((16,), 1, _i32)])
                al = a + b
                w = jnp.exp(jnp.maximum(al, 0.2 * al) - mv)
                plsc.store_scatter(wrows, [iota + g * 16, jnp.full((16,), 0, _i32)], w)
            gather.wait()

            def scale(q, _):
                for u in range(8):
                    ev = zero16 + (q * 8 + u)
                    wv = plsc.load_gather(wrows, [ev, zero16])
                    for j in range(2):
                        col = iota + j * 16
                        rv = plsc.load_gather(rowbuf, [ev, col])
                        plsc.store_scatter(rowbuf, [ev, col], rv * wv)
                return 0

            lax.fori_loop(0, CB // 8, scale, 0)
            sc1 = pltpu.async_copy(rowbuf, acc_sp.at[dstv], sema, add=True)
            sc2 = pltpu.async_copy(wrows, den_sp.at[dstv], semb, add=True)
            sc1.wait()
            sc2.wait()
            return 0

        lax.fori_loop(0, NCH2, chunk, 0)
        plsc.subcore_barrier()
        pltpu.sync_copy(acc_sp.at[pl.ds(r0, ROWS_PER_TEC)],
                        acc_out.at[pl.ds(coff + r0, ROWS_PER_TEC)])
        pltpu.sync_copy(den_sp.at[pl.ds(r0, ROWS_PER_TEC)],
                        den_out.at[pl.ds(coff + r0, ROWS_PER_TEC)])

    pl.run_scoped(
        inner,
        pltpu.VMEM((NPAD, 8), _f32),
        pltpu.VMEM((CB,), _i32),
        pltpu.VMEM((CB,), _i32),
        pltpu.VMEM((CB, 32), _f32),
        pltpu.VMEM((CB, 16), _f32),
        pltpu.VMEM((16,), _f32),
        pltpu.SemaphoreType.DMA,
        pltpu.SemaphoreType.DMA,
        pltpu.SemaphoreType.DMA,
    )


def _sc_gat2(srcp, dstp, g2tab, att2, c2rep, zer32, zer16):
    vmesh = plsc.VectorSubcoreMesh(core_axis_name="c", subcore_axis_name="s")
    smesh = plsc.ScalarSubcoreMesh(axis_name="c")
    f = pl.kernel(
        [_sc2_scs, _sc2_tec],
        out_type=[
            jax.ShapeDtypeStruct((2 * NPAD, 32), _f32),
            jax.ShapeDtypeStruct((2 * NPAD, 16), _f32),
        ],
        mesh=[smesh, vmesh],
        compiler_params=pltpu.CompilerParams(needs_layout_passes=False, use_tc_tiling_on_sc=False),
        scratch_types=[
            pltpu.VMEM_SHARED((NPAD, 32), _f32),
            pltpu.VMEM_SHARED((NPAD, 16), _f32),
        ],
    )
    return f(srcp, dstp, g2tab, att2, c2rep, zer32, zer16)


def kernel(x, edge_index, W_in, b_in, W1, att_src1, att_dst1, b1, Wr1, br1, g1, be1,
           W2, att_src2, att_dst2, b2, Wr2, br2, g2, be2, Wout, bout):
    v1 = (W_in @ W1)[0]
    c1 = b_in @ W1
    vr1 = (W_in @ Wr1)[0]
    cr1 = b_in @ Wr1 + br1
    consts = jnp.stack([v1, c1, vr1, cr1])
    ps = (v1.reshape(8, 32) * att_src1[0]).sum(-1)
    qs = (c1.reshape(8, 32) * att_src1[0]).sum(-1)
    pd = (v1.reshape(8, 32) * att_dst1[0]).sum(-1)
    qd = (c1.reshape(8, 32) * att_dst1[0]).sum(-1)
    attc = jnp.stack([jnp.concatenate([ps, pd]), jnp.concatenate([qs, qd])])

    xp = jnp.concatenate([x, jnp.zeros((NPAD - N, 1), _f32)], axis=0)

    loop = jnp.arange(N, dtype=_i32)
    padi = jnp.full((EPAD - ETOT,), N, _i32)
    srcp = jnp.concatenate([edge_index[0].astype(_i32), loop, padi])
    dstp = jnp.concatenate([edge_index[1].astype(_i32), loop, padi])

    h1, res1, att1, bmax = _stage_a(xp, consts, attc)

    ms = jnp.max(bmax, axis=(0, 1))
    cshift = _lrelu(ms[:8] + ms[8:])
    crep = jnp.repeat(cshift, 16)
    hcat = jnp.concatenate([h1[:, :128], h1[:, 128:]], axis=0)
    zpad8 = jnp.zeros((NPAD, 8), _f32)
    att16 = jnp.concatenate([
        jnp.concatenate([att1[:, 0:4], att1[:, 8:12], zpad8], axis=1),
        jnp.concatenate([att1[:, 4:8], att1[:, 12:16], zpad8], axis=1),
    ], axis=0)
    zer128 = jnp.zeros((NPAD, 128), _f32)
    zer16 = jnp.zeros((NPAD, 16), _f32)

    acc1, den1 = _sc_gat1(srcp, dstp, hcat, att16, crep, zer128, zer16)

    den8 = jnp.concatenate([den1[:NPAD, 0:4], den1[NPAD:, 0:4]], axis=1)
    denr = jnp.repeat(den8, 32, axis=1)
    cvec = jnp.stack([b1, g1, be1])
    c32 = jnp.stack([att_src2[0, 0], att_dst2[0, 0], br2])
    g2tab, res2, att2, bmax2 = _stage_c(acc1[:NPAD], acc1[NPAD:], denr, res1,
                                        cvec, W2, Wr2, c32)

    m2 = jnp.max(bmax2, axis=(0, 1))
    c2 = _lrelu(m2[0] + m2[1])
    c2rep = jnp.full((16,), c2, _f32)
    zer32 = jnp.zeros((NPAD, 32), _f32)

    acc2, den2 = _sc_gat2(srcp, dstp, g2tab, att2, c2rep, zer32, zer16)

    c32e = jnp.stack([b2, g2, be2])
    psum = _stage_e(acc2[:NPAD], acc2[NPAD:], den2[:NPAD], den2[NPAD:], res2, c32e)[0]
    pooled = jnp.sum(psum, axis=(0, 1)).reshape(1, 32) / N
    return pooled @ Wout + bout


# speedup vs baseline: 23.6491x; 1.1206x over previous
import functools

import jax
import jax.numpy as jnp
from jax import lax
from jax.experimental import pallas as pl
from jax.experimental.pallas import tpu as pltpu
from jax.experimental.pallas import tpu_sc as plsc

N = 10000
NPAD = 10240
E = 160000
ETOT = E + N
EPAD = 172032
BLK = 512
NB = NPAD // BLK
CB = 96
STG = 14
PER_TEC1 = EPAD // 16
PER_TEC2 = EPAD // 32
NCH1 = PER_TEC1 // CB
NCH2 = PER_TEC2 // CB
ROWS_PER_TEC = NPAD // 16

_i32 = jnp.int32
_f32 = jnp.float32


def _lrelu(x):
    return jnp.maximum(x, 0.2 * x)


def _stage_a_body(x_ref, consts_ref, attc_ref, h1_ref, res1_ref, att1_ref, bmax_ref):
    xv = x_ref[:, :]
    h1_ref[:, :] = xv * consts_ref[0:1, :] + consts_ref[1:2, :]
    res1_ref[:, :] = xv * consts_ref[2:3, :] + consts_ref[3:4, :]
    att = xv * attc_ref[0:1, :] + attc_ref[1:2, :]
    att1_ref[:, :] = att
    bmax_ref[0, :, :] = jnp.max(att, axis=0, keepdims=True)


def _stage_a(xp, consts, attc):
    return pl.pallas_call(
        _stage_a_body,
        grid=(NB,),
        in_specs=[
            pl.BlockSpec((BLK, 1), lambda i: (i, 0)),
            pl.BlockSpec((4, 256), lambda i: (0, 0)),
            pl.BlockSpec((2, 16), lambda i: (0, 0)),
        ],
        out_specs=[
            pl.BlockSpec((BLK, 256), lambda i: (i, 0)),
            pl.BlockSpec((BLK, 256), lambda i: (i, 0)),
            pl.BlockSpec((BLK, 16), lambda i: (i, 0)),
            pl.BlockSpec((1, 1, 16), lambda i: (i, 0, 0)),
        ],
        out_shape=[
            jax.ShapeDtypeStruct((NPAD, 256), _f32),
            jax.ShapeDtypeStruct((NPAD, 256), _f32),
            jax.ShapeDtypeStruct((NPAD, 16), _f32),
            jax.ShapeDtypeStruct((NB, 1, 16), _f32),
        ],
    )(xp, consts, attc)


def _stage_c_body(acca_ref, accb_ref, denr_ref, res1_ref, cvec_ref, w2_ref,
                  wr2_ref, c32_ref, g2_ref, res2_ref, att2_ref, bmax2_ref):
    acc = jnp.concatenate([acca_ref[:, :], accb_ref[:, :]], axis=1)
    o = acc / (denr_ref[:, :] + 1e-16) + cvec_ref[0:1, :]
    o = jnp.where(o > 0, o, jnp.exp(jnp.minimum(o, 0.0)) - 1.0)
    t = o + res1_ref[:, :]
    m = jnp.mean(t, axis=-1, keepdims=True)
    v = jnp.mean((t - m) ** 2, axis=-1, keepdims=True)
    h = (t - m) * lax.rsqrt(v + 1e-5) * cvec_ref[1:2, :] + cvec_ref[2:3, :]
    g2 = jnp.dot(h, w2_ref[:, :], preferred_element_type=_f32)
    g2_ref[:, :] = g2
    res2_ref[:, :] = jnp.dot(h, wr2_ref[:, :], preferred_element_type=_f32) + c32_ref[2:3, :]
    asrc2 = jnp.sum(g2 * c32_ref[0:1, :], axis=-1, keepdims=True)
    adst2 = jnp.sum(g2 * c32_ref[1:2, :], axis=-1, keepdims=True)
    lane = lax.broadcasted_iota(_i32, (BLK, 8), 1)
    att2 = jnp.where(lane == 0, asrc2, jnp.where(lane == 1, adst2, 0.0))
    att2_ref[:, :] = att2
    bmax2_ref[0, :, :] = jnp.max(att2, axis=0, keepdims=True)


def _stage_c(acca, accb, denr, res1, cvec, w2, wr2, c32):
    return pl.pallas_call(
        _stage_c_body,
        grid=(NB,),
        in_specs=[
            pl.BlockSpec((BLK, 128), lambda i: (i, 0)),
            pl.BlockSpec((BLK, 128), lambda i: (i, 0)),
            pl.BlockSpec((BLK, 256), lambda i: (i, 0)),
            pl.BlockSpec((BLK, 256), lambda i: (i, 0)),
            pl.BlockSpec((3, 256), lambda i: (0, 0)),
            pl.BlockSpec((256, 32), lambda i: (0, 0)),
            pl.BlockSpec((256, 32), lambda i: (0, 0)),
            pl.BlockSpec((3, 32), lambda i: (0, 0)),
        ],
        out_specs=[
            pl.BlockSpec((BLK, 32), lambda i: (i, 0)),
            pl.BlockSpec((BLK, 32), lambda i: (i, 0)),
            pl.BlockSpec((BLK, 8), lambda i: (i, 0)),
            pl.BlockSpec((1, 1, 8), lambda i: (i, 0, 0)),
        ],
        out_shape=[
            jax.ShapeDtypeStruct((NPAD, 32), _f32),
            jax.ShapeDtypeStruct((NPAD, 32), _f32),
            jax.ShapeDtypeStruct((NPAD, 8), _f32),
            jax.ShapeDtypeStruct((NB, 1, 8), _f32),
        ],
    )(acca, accb, denr, res1, cvec, w2, wr2, c32)


def _stage_e_body(a0_ref, a1_ref, d0_ref, d1_ref, res2_ref, c32_ref, psum_ref):
    i = pl.program_id(0)
    den = d0_ref[:, 0:1] + d1_ref[:, 0:1]
    o = (a0_ref[:, :] + a1_ref[:, :]) / (den + 1e-16) + c32_ref[0:1, :] + res2_ref[:, :]
    m = jnp.mean(o, axis=-1, keepdims=True)
    v = jnp.mean((o - m) ** 2, axis=-1, keepdims=True)
    hf = (o - m) * lax.rsqrt(v + 1e-5) * c32_ref[1:2, :] + c32_ref[2:3, :]
    rid = i * BLK + lax.broadcasted_iota(_i32, (BLK, 1), 0)
    hf = jnp.where(rid < N, hf, 0.0)
    psum_ref[0, :, :] = jnp.sum(hf, axis=0, keepdims=True)


def _stage_e(a0, a1, d0, d1, res2, c32):
    return pl.pallas_call(
        _stage_e_body,
        grid=(NB,),
        in_specs=[
            pl.BlockSpec((BLK, 32), lambda i: (i, 0)),
            pl.BlockSpec((BLK, 32), lambda i: (i, 0)),
            pl.BlockSpec((BLK, 16), lambda i: (i, 0)),
            pl.BlockSpec((BLK, 16), lambda i: (i, 0)),
            pl.BlockSpec((BLK, 32), lambda i: (i, 0)),
            pl.BlockSpec((3, 32), lambda i: (0, 0)),
        ],
        out_specs=[pl.BlockSpec((1, 1, 32), lambda i: (i, 0, 0))],
        out_shape=[jax.ShapeDtypeStruct((NB, 1, 32), _f32)],
    )(a0, a1, d0, d1, res2, c32)


def _sc1_scs(srcp, dstp, hcat, att16, crep, zer128, zer16,
             acc_out, den_out, acc_sp, den_sp):
    pass


def _sc1_tec(srcp, dstp, hcat, att16, crep, zer128, zer16,
             acc_out, den_out, acc_sp, den_sp):
    c = lax.axis_index("c")
    s = lax.axis_index("s")
    iota = lax.iota(_i32, 16)
    r0 = s * ROWS_PER_TEC
    coff = c * NPAD

    def inner(srcstage, dststage, crep_vm, *bufs):
        slots = [bufs[0:12], bufs[12:24]]
        pltpu.sync_copy(crep, crep_vm)
        mv = [plsc.load_gather(crep_vm, [iota + (c * 4 + h) * 16]) for h in range(4)]

        pltpu.sync_copy(zer128.at[pl.ds(r0, ROWS_PER_TEC)], acc_sp.at[pl.ds(r0, ROWS_PER_TEC)])
        pltpu.sync_copy(zer16.at[pl.ds(r0, ROWS_PER_TEC)], den_sp.at[pl.ds(r0, ROWS_PER_TEC)])
        for x in range(2):
            pltpu.sync_copy(zer16.at[pl.ds(0, CB)], slots[x][6])
        plsc.subcore_barrier()

        def prep(j, slot):
            srcoff, dstoff, dstv = slot[0], slot[1], slot[2]
            jj = jnp.full((16,), 0, _i32) + j
            for g in range(CB // 16):
                col = iota + g * 16
                sv = plsc.load_gather(srcstage, [jj, col])
                dv = plsc.load_gather(dststage, [jj, col])
                srcoff[pl.ds(g * 16, 16)] = sv + coff
                dstoff[pl.ds(g * 16, 16)] = dv + coff
                dstv[pl.ds(g * 16, 16)] = dv

        def issue(slot):
            grow = pltpu.async_copy(hcat.at[slot[0]], slot[5], slot[7])
            ga = pltpu.async_copy(att16.at[slot[0]], slot[3], slot[8])
            gb = pltpu.async_copy(att16.at[slot[1]], slot[4], slot[9])
            return grow, ga, gb

        def compute(slot, grow, ga, gb):
            abufs, abufd, rowbuf, wrows = slot[3], slot[4], slot[5], slot[6]
            ga.wait()
            gb.wait()
            for g in range(CB // 16):
                ridx = iota + g * 16
                for h in range(4):
                    hh = jnp.full((16,), h, _i32)
                    a = plsc.load_gather(abufs, [ridx, hh])
                    b = plsc.load_gather(abufd, [ridx, hh + 4])
                    al = a + b
                    w = jnp.exp(jnp.maximum(al, 0.2 * al) - mv[h])
                    plsc.store_scatter(wrows, [ridx, hh], w)
            grow.wait()

            def scale(q, _):
                for u in range(4):
                    ev = jnp.full((16,), 0, _i32) + (q * 4 + u)
                    wv = [plsc.load_gather(wrows, [ev, jnp.full((16,), h, _i32)])
                          for h in range(4)]
                    for h in range(4):
                        for j in range(2):
                            col = iota + (h * 32 + j * 16)
                            rv = plsc.load_gather(rowbuf, [ev, col])
                            plsc.store_scatter(rowbuf, [ev, col], rv * wv[h])
                return 0

            lax.fori_loop(0, CB // 4, scale, 0)
            sa = pltpu.async_copy(rowbuf, acc_sp.at[slot[2]], slot[10], add=True)
            sd = pltpu.async_copy(wrows, den_sp.at[slot[2]], slot[11], add=True)
            return sa, sd

        def body2(k, _):
            @pl.when(lax.rem(k, STG // 2) == 0)
            def _refill():
                rbase = s * NCH1 + (k // (STG // 2)) * STG
                pltpu.sync_copy(srcp.at[pl.ds(rbase, STG)], srcstage)
                pltpu.sync_copy(dstp.at[pl.ds(rbase, STG)], dststage)

            j0 = lax.rem(2 * k, STG)
            prep(j0, slots[0])
            prep(j0 + 1, slots[1])
            g0 = issue(slots[0])
            g1 = issue(slots[1])
            s0 = compute(slots[0], *g0)
            s1 = compute(slots[1], *g1)
            s0[0].wait()
            s0[1].wait()
            s1[0].wait()
            s1[1].wait()
            return 0

        lax.fori_loop(0, NCH1 // 2, body2, 0)
        plsc.subcore_barrier()
        pltpu.sync_copy(acc_sp.at[pl.ds(r0, ROWS_PER_TEC)],
                        acc_out.at[pl.ds(coff + r0, ROWS_PER_TEC)])
        pltpu.sync_copy(den_sp.at[pl.ds(r0, ROWS_PER_TEC)],
                        den_out.at[pl.ds(coff + r0, ROWS_PER_TEC)])

    slot_types = [
        pltpu.VMEM((CB,), _i32),
        pltpu.VMEM((CB,), _i32),
        pltpu.VMEM((CB,), _i32),
        pltpu.VMEM((CB, 16), _f32),
        pltpu.VMEM((CB, 16), _f32),
        pltpu.VMEM((CB, 128), _f32),
        pltpu.VMEM((CB, 16), _f32),
        pltpu.SemaphoreType.DMA,
        pltpu.SemaphoreType.DMA,
        pltpu.SemaphoreType.DMA,
        pltpu.SemaphoreType.DMA,
        pltpu.SemaphoreType.DMA,
    ]
    pl.run_scoped(
        inner,
        pltpu.VMEM((STG, CB), _i32),
        pltpu.VMEM((STG, CB), _i32),
        pltpu.VMEM((128,), _f32),
        *(slot_types * 2),
    )


def _sc_gat1(srcp, dstp, hcat, att16, crep, zer128, zer16):
    vmesh = plsc.VectorSubcoreMesh(core_axis_name="c", subcore_axis_name="s")
    smesh = plsc.ScalarSubcoreMesh(axis_name="c")
    f = pl.kernel(
        [_sc1_scs, _sc1_tec],
        out_type=[
            jax.ShapeDtypeStruct((2 * NPAD, 128), _f32),
            jax.ShapeDtypeStruct((2 * NPAD, 16), _f32),
        ],
        mesh=[smesh, vmesh],
        compiler_params=pltpu.CompilerParams(needs_layout_passes=False, use_tc_tiling_on_sc=False),
        scratch_types=[
            pltpu.VMEM_SHARED((NPAD, 128), _f32),
            pltpu.VMEM_SHARED((NPAD, 16), _f32),
        ],
    )
    return f(srcp, dstp, hcat, att16, crep, zer128, zer16)


def _sc2_scs(srcp, dstp, g2tab, att2, c2rep, zer32, zer16,
             acc_out, den_out, acc_sp, den_sp):
    pass


def _sc2_tec(srcp, dstp, g2tab, att2, c2rep, zer32, zer16,
             acc_out, den_out, acc_sp, den_sp):
    c = lax.axis_index("c")
    s = lax.axis_index("s")
    iota = lax.iota(_i32, 16)
    zero16 = jnp.full((16,), 0, _i32)
    r0 = s * ROWS_PER_TEC
    ebase = (c * 16 + s) * PER_TEC2
    coff = c * NPAD

    def inner(att_vm, srcv, dstv, rowbuf, wrows, c2vm, sem, sema, semb):
        pltpu.sync_copy(att2, att_vm)
        pltpu.sync_copy(c2rep, c2vm)
        mv = c2vm[pl.ds(0, 16)]

        pltpu.sync_copy(zer32.at[pl.ds(r0, ROWS_PER_TEC)], acc_sp.at[pl.ds(r0, ROWS_PER_TEC)])
        pltpu.sync_copy(zer16.at[pl.ds(r0, ROWS_PER_TEC)], den_sp.at[pl.ds(r0, ROWS_PER_TEC)])
        pltpu.sync_copy(zer16.at[pl.ds(0, CB)], wrows)
        plsc.subcore_barrier()

        def chunk(i, _):
            base = ebase + i * CB
            pltpu.sync_copy(srcp.at[pl.ds(base, CB)], srcv)
            pltpu.sync_copy(dstp.at[pl.ds(base, CB)], dstv)
            gather = pltpu.async_copy(g2tab.at[srcv], rowbuf, sem)
            for g in range(CB // 16):
                sidx = srcv[pl.ds(g * 16, 16)]
                didx = dstv[pl.ds(g * 16, 16)]
                a = plsc.load_gather(att_vm, [sidx, jnp.full((16,), 0, _i32)])
                b = plsc.load_gather(att_vm, [didx, jnp.full((16,), 1, _i32)])
                al = a + b
                w = jnp.exp(jnp.maximum(al, 0.2 * al) - mv)
                plsc.store_scatter(wrows, [iota + g * 16, jnp.full((16,), 0, _i32)], w)
            gather.wait()

            def scale(q, _):
                for u in range(8):
                    ev = zero16 + (q * 8 + u)
                    wv = plsc.load_gather(wrows, [ev, zero16])
                    for j in range(2):
                        col = iota + j * 16
                        rv = plsc.load_gather(rowbuf, [ev, col])
                        plsc.store_scatter(rowbuf, [ev, col], rv * wv)
                return 0

            lax.fori_loop(0, CB // 8, scale, 0)
            sc1 = pltpu.async_copy(rowbuf, acc_sp.at[dstv], sema, add=True)
            sc2 = pltpu.async_copy(wrows, den_sp.at[dstv], semb, add=True)
            sc1.wait()
            sc2.wait()
            return 0

        lax.fori_loop(0, NCH2, chunk, 0)
        plsc.subcore_barrier()
        pltpu.sync_copy(acc_sp.at[pl.ds(r0, ROWS_PER_TEC)],
                        acc_out.at[pl.ds(coff + r0, ROWS_PER_TEC)])
        pltpu.sync_copy(den_sp.at[pl.ds(r0, ROWS_PER_TEC)],
                        den_out.at[pl.ds(coff + r0, ROWS_PER_TEC)])

    pl.run_scoped(
        inner,
        pltpu.VMEM((NPAD, 8), _f32),
        pltpu.VMEM((CB,), _i32),
        pltpu.VMEM((CB,), _i32),
        pltpu.VMEM((CB, 32), _f32),
        pltpu.VMEM((CB, 16), _f32),
        pltpu.VMEM((16,), _f32),
        pltpu.SemaphoreType.DMA,
        pltpu.SemaphoreType.DMA,
        pltpu.SemaphoreType.DMA,
    )


def _sc_gat2(srcp, dstp, g2tab, att2, c2rep, zer32, zer16):
    vmesh = plsc.VectorSubcoreMesh(core_axis_name="c", subcore_axis_name="s")
    smesh = plsc.ScalarSubcoreMesh(axis_name="c")
    f = pl.kernel(
        [_sc2_scs, _sc2_tec],
        out_type=[
            jax.ShapeDtypeStruct((2 * NPAD, 32), _f32),
            jax.ShapeDtypeStruct((2 * NPAD, 16), _f32),
        ],
        mesh=[smesh, vmesh],
        compiler_params=pltpu.CompilerParams(needs_layout_passes=False, use_tc_tiling_on_sc=False),
        scratch_types=[
            pltpu.VMEM_SHARED((NPAD, 32), _f32),
            pltpu.VMEM_SHARED((NPAD, 16), _f32),
        ],
    )
    return f(srcp, dstp, g2tab, att2, c2rep, zer32, zer16)


def kernel(x, edge_index, W_in, b_in, W1, att_src1, att_dst1, b1, Wr1, br1, g1, be1,
           W2, att_src2, att_dst2, b2, Wr2, br2, g2, be2, Wout, bout):
    v1 = (W_in @ W1)[0]
    c1 = b_in @ W1
    vr1 = (W_in @ Wr1)[0]
    cr1 = b_in @ Wr1 + br1
    consts = jnp.stack([v1, c1, vr1, cr1])
    ps = (v1.reshape(8, 32) * att_src1[0]).sum(-1)
    qs = (c1.reshape(8, 32) * att_src1[0]).sum(-1)
    pd = (v1.reshape(8, 32) * att_dst1[0]).sum(-1)
    qd = (c1.reshape(8, 32) * att_dst1[0]).sum(-1)
    attc = jnp.stack([jnp.concatenate([ps, pd]), jnp.concatenate([qs, qd])])

    xp = jnp.concatenate([x, jnp.zeros((NPAD - N, 1), _f32)], axis=0)

    loop = jnp.arange(N, dtype=_i32)
    padi = jnp.full((EPAD - ETOT,), N, _i32)
    srcp = jnp.concatenate([edge_index[0].astype(_i32), loop, padi])
    dstp = jnp.concatenate([edge_index[1].astype(_i32), loop, padi])
    srcp2 = srcp.reshape(EPAD // CB, CB)
    dstp2 = dstp.reshape(EPAD // CB, CB)

    h1, res1, att1, bmax = _stage_a(xp, consts, attc)

    ms = jnp.max(bmax, axis=(0, 1))
    cshift = _lrelu(ms[:8] + ms[8:])
    crep = jnp.repeat(cshift, 16)
    hcat = jnp.concatenate([h1[:, :128], h1[:, 128:]], axis=0)
    zpad8 = jnp.zeros((NPAD, 8), _f32)
    att16 = jnp.concatenate([
        jnp.concatenate([att1[:, 0:4], att1[:, 8:12], zpad8], axis=1),
        jnp.concatenate([att1[:, 4:8], att1[:, 12:16], zpad8], axis=1),
    ], axis=0)
    zer128 = jnp.zeros((NPAD, 128), _f32)
    zer16 = jnp.zeros((NPAD, 16), _f32)

    acc1, den1 = _sc_gat1(srcp2, dstp2, hcat, att16, crep, zer128, zer16)

    den8 = jnp.concatenate([den1[:NPAD, 0:4], den1[NPAD:, 0:4]], axis=1)
    denr = jnp.repeat(den8, 32, axis=1)
    cvec = jnp.stack([b1, g1, be1])
    c32 = jnp.stack([att_src2[0, 0], att_dst2[0, 0], br2])
    g2tab, res2, att2, bmax2 = _stage_c(acc1[:NPAD], acc1[NPAD:], denr, res1,
                                        cvec, W2, Wr2, c32)

    m2 = jnp.max(bmax2, axis=(0, 1))
    c2 = _lrelu(m2[0] + m2[1])
    c2rep = jnp.full((16,), c2, _f32)
    zer32 = jnp.zeros((NPAD, 32), _f32)

    acc2, den2 = _sc_gat2(srcp, dstp, g2tab, att2, c2rep, zer32, zer16)

    c32e = jnp.stack([b2, g2, be2])
    psum = _stage_e(acc2[:NPAD], acc2[NPAD:], den2[:NPAD], den2[NPAD:], res2, c32e)[0]
    pooled = jnp.sum(psum, axis=(0, 1)).reshape(1, 32) / N
    return pooled @ Wout + bout


# speedup vs baseline: 24.8774x; 1.0519x over previous
import functools

import jax
import jax.numpy as jnp
from jax import lax
from jax.experimental import pallas as pl
from jax.experimental.pallas import tpu as pltpu
from jax.experimental.pallas import tpu_sc as plsc

N = 10000
NPAD = 10240
E = 160000
ETOT = E + N
EPAD = 172032
BLK = 512
NB = NPAD // BLK
CB = 96
STG = 14
PER_TEC1 = EPAD // 16
PER_TEC2 = EPAD // 32
NCH1 = PER_TEC1 // CB
NCH2 = PER_TEC2 // CB
ROWS_PER_TEC = NPAD // 16

_i32 = jnp.int32
_f32 = jnp.float32


def _lrelu(x):
    return jnp.maximum(x, 0.2 * x)


def _stage_a_body(x_ref, consts_ref, attc_ref, h1_ref, res1_ref, att1_ref, bmax_ref):
    xv = x_ref[:, :]
    h1_ref[:, :] = xv * consts_ref[0:1, :] + consts_ref[1:2, :]
    res1_ref[:, :] = xv * consts_ref[2:3, :] + consts_ref[3:4, :]
    att = xv * attc_ref[0:1, :] + attc_ref[1:2, :]
    att1_ref[:, :] = att
    bmax_ref[0, :, :] = jnp.max(att, axis=0, keepdims=True)


def _stage_a(xp, consts, attc):
    return pl.pallas_call(
        _stage_a_body,
        grid=(NB,),
        in_specs=[
            pl.BlockSpec((BLK, 1), lambda i: (i, 0)),
            pl.BlockSpec((4, 256), lambda i: (0, 0)),
            pl.BlockSpec((2, 16), lambda i: (0, 0)),
        ],
        out_specs=[
            pl.BlockSpec((BLK, 256), lambda i: (i, 0)),
            pl.BlockSpec((BLK, 256), lambda i: (i, 0)),
            pl.BlockSpec((BLK, 16), lambda i: (i, 0)),
            pl.BlockSpec((1, 1, 16), lambda i: (i, 0, 0)),
        ],
        out_shape=[
            jax.ShapeDtypeStruct((NPAD, 256), _f32),
            jax.ShapeDtypeStruct((NPAD, 256), _f32),
            jax.ShapeDtypeStruct((NPAD, 16), _f32),
            jax.ShapeDtypeStruct((NB, 1, 16), _f32),
        ],
    )(xp, consts, attc)


def _stage_c_body(acca_ref, accb_ref, denr_ref, res1_ref, cvec_ref, w2_ref,
                  wr2_ref, c32_ref, g2_ref, res2_ref, att2_ref, bmax2_ref):
    acc = jnp.concatenate([acca_ref[:, :], accb_ref[:, :]], axis=1)
    o = acc / (denr_ref[:, :] + 1e-16) + cvec_ref[0:1, :]
    o = jnp.where(o > 0, o, jnp.exp(jnp.minimum(o, 0.0)) - 1.0)
    t = o + res1_ref[:, :]
    m = jnp.mean(t, axis=-1, keepdims=True)
    v = jnp.mean((t - m) ** 2, axis=-1, keepdims=True)
    h = (t - m) * lax.rsqrt(v + 1e-5) * cvec_ref[1:2, :] + cvec_ref[2:3, :]
    g2 = jnp.dot(h, w2_ref[:, :], preferred_element_type=_f32)
    g2_ref[:, :] = g2
    res2_ref[:, :] = jnp.dot(h, wr2_ref[:, :], preferred_element_type=_f32) + c32_ref[2:3, :]
    asrc2 = jnp.sum(g2 * c32_ref[0:1, :], axis=-1, keepdims=True)
    adst2 = jnp.sum(g2 * c32_ref[1:2, :], axis=-1, keepdims=True)
    lane = lax.broadcasted_iota(_i32, (BLK, 8), 1)
    att2 = jnp.where(lane == 0, asrc2, jnp.where(lane == 1, adst2, 0.0))
    att2_ref[:, :] = att2
    bmax2_ref[0, :, :] = jnp.max(att2, axis=0, keepdims=True)


def _stage_c(acca, accb, denr, res1, cvec, w2, wr2, c32):
    return pl.pallas_call(
        _stage_c_body,
        grid=(NB,),
        in_specs=[
            pl.BlockSpec((BLK, 128), lambda i: (i, 0)),
            pl.BlockSpec((BLK, 128), lambda i: (i, 0)),
            pl.BlockSpec((BLK, 256), lambda i: (i, 0)),
            pl.BlockSpec((BLK, 256), lambda i: (i, 0)),
            pl.BlockSpec((3, 256), lambda i: (0, 0)),
            pl.BlockSpec((256, 32), lambda i: (0, 0)),
            pl.BlockSpec((256, 32), lambda i: (0, 0)),
            pl.BlockSpec((3, 32), lambda i: (0, 0)),
        ],
        out_specs=[
            pl.BlockSpec((BLK, 32), lambda i: (i, 0)),
            pl.BlockSpec((BLK, 32), lambda i: (i, 0)),
            pl.BlockSpec((BLK, 8), lambda i: (i, 0)),
            pl.BlockSpec((1, 1, 8), lambda i: (i, 0, 0)),
        ],
        out_shape=[
            jax.ShapeDtypeStruct((NPAD, 32), _f32),
            jax.ShapeDtypeStruct((NPAD, 32), _f32),
            jax.ShapeDtypeStruct((NPAD, 8), _f32),
            jax.ShapeDtypeStruct((NB, 1, 8), _f32),
        ],
    )(acca, accb, denr, res1, cvec, w2, wr2, c32)


def _stage_e_body(a0_ref, a1_ref, d0_ref, d1_ref, res2_ref, c32_ref, psum_ref):
    i = pl.program_id(0)
    den = d0_ref[:, 0:1] + d1_ref[:, 0:1]
    o = (a0_ref[:, :] + a1_ref[:, :]) / (den + 1e-16) + c32_ref[0:1, :] + res2_ref[:, :]
    m = jnp.mean(o, axis=-1, keepdims=True)
    v = jnp.mean((o - m) ** 2, axis=-1, keepdims=True)
    hf = (o - m) * lax.rsqrt(v + 1e-5) * c32_ref[1:2, :] + c32_ref[2:3, :]
    rid = i * BLK + lax.broadcasted_iota(_i32, (BLK, 1), 0)
    hf = jnp.where(rid < N, hf, 0.0)
    psum_ref[0, :, :] = jnp.sum(hf, axis=0, keepdims=True)


def _stage_e(a0, a1, d0, d1, res2, c32):
    return pl.pallas_call(
        _stage_e_body,
        grid=(NB,),
        in_specs=[
            pl.BlockSpec((BLK, 32), lambda i: (i, 0)),
            pl.BlockSpec((BLK, 32), lambda i: (i, 0)),
            pl.BlockSpec((BLK, 16), lambda i: (i, 0)),
            pl.BlockSpec((BLK, 16), lambda i: (i, 0)),
            pl.BlockSpec((BLK, 32), lambda i: (i, 0)),
            pl.BlockSpec((3, 32), lambda i: (0, 0)),
        ],
        out_specs=[pl.BlockSpec((1, 1, 32), lambda i: (i, 0, 0))],
        out_shape=[jax.ShapeDtypeStruct((NB, 1, 32), _f32)],
    )(a0, a1, d0, d1, res2, c32)


def _sc1_scs(srcp, dstp, hcat, att16, crep, zer128, zer16,
             acc_out, den_out, acc_sp, den_sp):
    pass


def _sc1_tec(srcp, dstp, hcat, att16, crep, zer128, zer16,
             acc_out, den_out, acc_sp, den_sp):
    c = lax.axis_index("c")
    s = lax.axis_index("s")
    iota = lax.iota(_i32, 16)
    r0 = s * ROWS_PER_TEC
    coff = c * NPAD

    def inner(srcstage, dststage, crep_vm, *bufs):
        slots = [bufs[0:12], bufs[12:24]]
        pltpu.sync_copy(crep, crep_vm)
        mv = [plsc.load_gather(crep_vm, [iota + (c * 4 + h) * 16]) for h in range(4)]

        pltpu.sync_copy(zer128.at[pl.ds(r0, ROWS_PER_TEC)], acc_sp.at[pl.ds(r0, ROWS_PER_TEC)])
        pltpu.sync_copy(zer16.at[pl.ds(r0, ROWS_PER_TEC)], den_sp.at[pl.ds(r0, ROWS_PER_TEC)])
        for x in range(2):
            pltpu.sync_copy(zer16.at[pl.ds(0, CB)], slots[x][6])
        plsc.subcore_barrier()

        def prep(j, slot):
            srcoff, dstoff, dstv = slot[0], slot[1], slot[2]
            jj = jnp.full((16,), 0, _i32) + j
            for g in range(CB // 16):
                col = iota + g * 16
                sv = plsc.load_gather(srcstage, [jj, col])
                dv = plsc.load_gather(dststage, [jj, col])
                srcoff[pl.ds(g * 16, 16)] = sv + coff
                dstoff[pl.ds(g * 16, 16)] = dv + coff
                dstv[pl.ds(g * 16, 16)] = dv

        def issue(slot):
            grow = pltpu.async_copy(hcat.at[slot[0]], slot[5], slot[7])
            ga = pltpu.async_copy(att16.at[slot[0]], slot[3], slot[8])
            gb = pltpu.async_copy(att16.at[slot[1]], slot[4], slot[9])
            return grow, ga, gb

        def compute(slot, grow, ga, gb):
            abufs, abufd, rowbuf, wrows = slot[3], slot[4], slot[5], slot[6]
            ga.wait()
            gb.wait()
            for g in range(CB // 16):
                ridx = iota + g * 16
                for h in range(4):
                    hh = jnp.full((16,), h, _i32)
                    a = plsc.load_gather(abufs, [ridx, hh])
                    b = plsc.load_gather(abufd, [ridx, hh + 4])
                    al = a + b
                    w = jnp.exp(jnp.maximum(al, 0.2 * al) - mv[h])
                    plsc.store_scatter(wrows, [ridx, hh], w)
            grow.wait()

            def scale(q, _):
                for u in range(4):
                    ev = jnp.full((16,), 0, _i32) + (q * 4 + u)
                    wv = [plsc.load_gather(wrows, [ev, jnp.full((16,), h, _i32)])
                          for h in range(4)]
                    for h in range(4):
                        for j in range(2):
                            col = iota + (h * 32 + j * 16)
                            rv = plsc.load_gather(rowbuf, [ev, col])
                            plsc.store_scatter(rowbuf, [ev, col], rv * wv[h])
                return 0

            lax.fori_loop(0, CB // 4, scale, 0)
            sa = pltpu.async_copy(rowbuf, acc_sp.at[slot[2]], slot[10], add=True)
            sd = pltpu.async_copy(wrows, den_sp.at[slot[2]], slot[11], add=True)
            return sa, sd

        def body2(k, _):
            @pl.when(lax.rem(k, STG // 2) == 0)
            def _refill():
                rbase = s * NCH1 + (k // (STG // 2)) * STG
                pltpu.sync_copy(srcp.at[pl.ds(rbase, STG)], srcstage)
                pltpu.sync_copy(dstp.at[pl.ds(rbase, STG)], dststage)

            j0 = lax.rem(2 * k, STG)
            prep(j0, slots[0])
            prep(j0 + 1, slots[1])
            g0 = issue(slots[0])
            g1 = issue(slots[1])
            s0 = compute(slots[0], *g0)
            s1 = compute(slots[1], *g1)
            s0[0].wait()
            s0[1].wait()
            s1[0].wait()
            s1[1].wait()
            return 0

        lax.fori_loop(0, NCH1 // 2, body2, 0)
        plsc.subcore_barrier()
        pltpu.sync_copy(acc_sp.at[pl.ds(r0, ROWS_PER_TEC)],
                        acc_out.at[pl.ds(coff + r0, ROWS_PER_TEC)])
        pltpu.sync_copy(den_sp.at[pl.ds(r0, ROWS_PER_TEC)],
                        den_out.at[pl.ds(coff + r0, ROWS_PER_TEC)])

    slot_types = [
        pltpu.VMEM((CB,), _i32),
        pltpu.VMEM((CB,), _i32),
        pltpu.VMEM((CB,), _i32),
        pltpu.VMEM((CB, 16), _f32),
        pltpu.VMEM((CB, 16), _f32),
        pltpu.VMEM((CB, 128), _f32),
        pltpu.VMEM((CB, 16), _f32),
        pltpu.SemaphoreType.DMA,
        pltpu.SemaphoreType.DMA,
        pltpu.SemaphoreType.DMA,
        pltpu.SemaphoreType.DMA,
        pltpu.SemaphoreType.DMA,
    ]
    pl.run_scoped(
        inner,
        pltpu.VMEM((STG, CB), _i32),
        pltpu.VMEM((STG, CB), _i32),
        pltpu.VMEM((128,), _f32),
        *(slot_types * 2),
    )


def _sc_gat1(srcp, dstp, hcat, att16, crep, zer128, zer16):
    vmesh = plsc.VectorSubcoreMesh(core_axis_name="c", subcore_axis_name="s")
    smesh = plsc.ScalarSubcoreMesh(axis_name="c")
    f = pl.kernel(
        [_sc1_scs, _sc1_tec],
        out_type=[
            jax.ShapeDtypeStruct((2 * NPAD, 128), _f32),
            jax.ShapeDtypeStruct((2 * NPAD, 16), _f32),
        ],
        mesh=[smesh, vmesh],
        compiler_params=pltpu.CompilerParams(needs_layout_passes=False, use_tc_tiling_on_sc=False),
        scratch_types=[
            pltpu.VMEM_SHARED((NPAD, 128), _f32),
            pltpu.VMEM_SHARED((NPAD, 16), _f32),
        ],
    )
    return f(srcp, dstp, hcat, att16, crep, zer128, zer16)


def _sc2_scs(srcp, dstp, g2tab, att2, c2rep, zer32, zer16,
             acc_out, den_out, acc_sp, den_sp):
    pass


def _sc2_tec(srcp, dstp, g2tab, att2, c2rep, zer32, zer16,
             acc_out, den_out, acc_sp, den_sp):
    c = lax.axis_index("c")
    s = lax.axis_index("s")
    iota = lax.iota(_i32, 16)
    zero16 = jnp.full((16,), 0, _i32)
    r0 = s * ROWS_PER_TEC
    rowbase = (c * 16 + s) * NCH2
    coff = c * NPAD

    def inner(att_vm, srcstage, dststage, c2vm, *bufs):
        slots = [bufs[0:7], bufs[7:14]]
        pltpu.sync_copy(att2, att_vm)
        pltpu.sync_copy(c2rep, c2vm)
        mv = c2vm[pl.ds(0, 16)]

        pltpu.sync_copy(zer32.at[pl.ds(r0, ROWS_PER_TEC)], acc_sp.at[pl.ds(r0, ROWS_PER_TEC)])
        pltpu.sync_copy(zer16.at[pl.ds(r0, ROWS_PER_TEC)], den_sp.at[pl.ds(r0, ROWS_PER_TEC)])
        for x in range(2):
            pltpu.sync_copy(zer16.at[pl.ds(0, CB)], slots[x][3])
        plsc.subcore_barrier()

        def prep(j, slot):
            srcv, dstv = slot[0], slot[1]
            jj = jnp.full((16,), 0, _i32) + j
            for g in range(CB // 16):
                col = iota + g * 16
                sv = plsc.load_gather(srcstage, [jj, col])
                dv = plsc.load_gather(dststage, [jj, col])
                srcv[pl.ds(g * 16, 16)] = sv
                dstv[pl.ds(g * 16, 16)] = dv

        def issue(slot):
            return pltpu.async_copy(g2tab.at[slot[0]], slot[2], slot[4])

        def compute(slot, grow):
            srcv, dstv, rowbuf, wrows = slot[0], slot[1], slot[2], slot[3]
            for g in range(CB // 16):
                sidx = srcv[pl.ds(g * 16, 16)]
                didx = dstv[pl.ds(g * 16, 16)]
                a = plsc.load_gather(att_vm, [sidx, jnp.full((16,), 0, _i32)])
                b = plsc.load_gather(att_vm, [didx, jnp.full((16,), 1, _i32)])
                al = a + b
                w = jnp.exp(jnp.maximum(al, 0.2 * al) - mv)
                plsc.store_scatter(wrows, [iota + g * 16, jnp.full((16,), 0, _i32)], w)
            grow.wait()

            def scale(q, _):
                for u in range(8):
                    ev = zero16 + (q * 8 + u)
                    wv = plsc.load_gather(wrows, [ev, zero16])
                    for j in range(2):
                        col = iota + j * 16
                        rv = plsc.load_gather(rowbuf, [ev, col])
                        plsc.store_scatter(rowbuf, [ev, col], rv * wv)
                return 0

            lax.fori_loop(0, CB // 8, scale, 0)
            sa = pltpu.async_copy(rowbuf, acc_sp.at[slot[1]], slot[5], add=True)
            sd = pltpu.async_copy(wrows, den_sp.at[slot[1]], slot[6], add=True)
            return sa, sd

        def body2(k, _):
            @pl.when(lax.rem(k, STG // 2) == 0)
            def _refill():
                rb = rowbase + (k // (STG // 2)) * STG
                pltpu.sync_copy(srcp.at[pl.ds(rb, STG)], srcstage)
                pltpu.sync_copy(dstp.at[pl.ds(rb, STG)], dststage)

            j0 = lax.rem(2 * k, STG)
            prep(j0, slots[0])
            prep(j0 + 1, slots[1])
            g0 = issue(slots[0])
            g1 = issue(slots[1])
            s0 = compute(slots[0], g0)
            s1 = compute(slots[1], g1)
            s0[0].wait()
            s0[1].wait()
            s1[0].wait()
            s1[1].wait()
            return 0

        lax.fori_loop(0, NCH2 // 2, body2, 0)
        plsc.subcore_barrier()
        pltpu.sync_copy(acc_sp.at[pl.ds(r0, ROWS_PER_TEC)],
                        acc_out.at[pl.ds(coff + r0, ROWS_PER_TEC)])
        pltpu.sync_copy(den_sp.at[pl.ds(r0, ROWS_PER_TEC)],
                        den_out.at[pl.ds(coff + r0, ROWS_PER_TEC)])

    slot_types = [
        pltpu.VMEM((CB,), _i32),
        pltpu.VMEM((CB,), _i32),
        pltpu.VMEM((CB, 32), _f32),
        pltpu.VMEM((CB, 16), _f32),
        pltpu.SemaphoreType.DMA,
        pltpu.SemaphoreType.DMA,
        pltpu.SemaphoreType.DMA,
    ]
    pl.run_scoped(
        inner,
        pltpu.VMEM((NPAD, 8), _f32),
        pltpu.VMEM((STG, CB), _i32),
        pltpu.VMEM((STG, CB), _i32),
        pltpu.VMEM((16,), _f32),
        *(slot_types * 2),
    )


def _sc_gat2(srcp, dstp, g2tab, att2, c2rep, zer32, zer16):
    vmesh = plsc.VectorSubcoreMesh(core_axis_name="c", subcore_axis_name="s")
    smesh = plsc.ScalarSubcoreMesh(axis_name="c")
    f = pl.kernel(
        [_sc2_scs, _sc2_tec],
        out_type=[
            jax.ShapeDtypeStruct((2 * NPAD, 32), _f32),
            jax.ShapeDtypeStruct((2 * NPAD, 16), _f32),
        ],
        mesh=[smesh, vmesh],
        compiler_params=pltpu.CompilerParams(needs_layout_passes=False, use_tc_tiling_on_sc=False),
        scratch_types=[
            pltpu.VMEM_SHARED((NPAD, 32), _f32),
            pltpu.VMEM_SHARED((NPAD, 16), _f32),
        ],
    )
    return f(srcp, dstp, g2tab, att2, c2rep, zer32, zer16)


def kernel(x, edge_index, W_in, b_in, W1, att_src1, att_dst1, b1, Wr1, br1, g1, be1,
           W2, att_src2, att_dst2, b2, Wr2, br2, g2, be2, Wout, bout):
    v1 = (W_in @ W1)[0]
    c1 = b_in @ W1
    vr1 = (W_in @ Wr1)[0]
    cr1 = b_in @ Wr1 + br1
    consts = jnp.stack([v1, c1, vr1, cr1])
    ps = (v1.reshape(8, 32) * att_src1[0]).sum(-1)
    qs = (c1.reshape(8, 32) * att_src1[0]).sum(-1)
    pd = (v1.reshape(8, 32) * att_dst1[0]).sum(-1)
    qd = (c1.reshape(8, 32) * att_dst1[0]).sum(-1)
    attc = jnp.stack([jnp.concatenate([ps, pd]), jnp.concatenate([qs, qd])])

    xp = jnp.concatenate([x, jnp.zeros((NPAD - N, 1), _f32)], axis=0)

    loop = jnp.arange(N, dtype=_i32)
    padi = jnp.full((EPAD - ETOT,), N, _i32)
    srcp = jnp.concatenate([edge_index[0].astype(_i32), loop, padi])
    dstp = jnp.concatenate([edge_index[1].astype(_i32), loop, padi])
    srcp2 = srcp.reshape(EPAD // CB, CB)
    dstp2 = dstp.reshape(EPAD // CB, CB)

    h1, res1, att1, bmax = _stage_a(xp, consts, attc)

    ms = jnp.max(bmax, axis=(0, 1))
    cshift = _lrelu(ms[:8] + ms[8:])
    crep = jnp.repeat(cshift, 16)
    hcat = jnp.concatenate([h1[:, :128], h1[:, 128:]], axis=0)
    zpad8 = jnp.zeros((NPAD, 8), _f32)
    att16 = jnp.concatenate([
        jnp.concatenate([att1[:, 0:4], att1[:, 8:12], zpad8], axis=1),
        jnp.concatenate([att1[:, 4:8], att1[:, 12:16], zpad8], axis=1),
    ], axis=0)
    zer128 = jnp.zeros((NPAD, 128), _f32)
    zer16 = jnp.zeros((NPAD, 16), _f32)

    acc1, den1 = _sc_gat1(srcp2, dstp2, hcat, att16, crep, zer128, zer16)

    den8 = jnp.concatenate([den1[:NPAD, 0:4], den1[NPAD:, 0:4]], axis=1)
    denr = jnp.repeat(den8, 32, axis=1)
    cvec = jnp.stack([b1, g1, be1])
    c32 = jnp.stack([att_src2[0, 0], att_dst2[0, 0], br2])
    g2tab, res2, att2, bmax2 = _stage_c(acc1[:NPAD], acc1[NPAD:], denr, res1,
                                        cvec, W2, Wr2, c32)

    m2 = jnp.max(bmax2, axis=(0, 1))
    c2 = _lrelu(m2[0] + m2[1])
    c2rep = jnp.full((16,), c2, _f32)
    zer32 = jnp.zeros((NPAD, 32), _f32)

    acc2, den2 = _sc_gat2(srcp2, dstp2, g2tab, att2, c2rep, zer32, zer16)

    c32e = jnp.stack([b2, g2, be2])
    psum = _stage_e(acc2[:NPAD], acc2[NPAD:], den2[:NPAD], den2[NPAD:], res2, c32e)[0]
    pooled = jnp.sum(psum, axis=(0, 1)).reshape(1, 32) / N
    return pooled @ Wout + bout
